# baseline (speedup 1.0000x reference)
import functools

import jax
import jax.numpy as jnp
from jax import lax
from jax.experimental import pallas as pl
from jax.experimental.pallas import tpu as pltpu
from jax.experimental.pallas import tpu_sc as plsc

_N = 10000
_E = 320000
_DIM = 128
_LANES = 16
_NC = 2
_NS = 16
_NW = _NC * _NS
_EPW = _E // _NW


_UNROLL = 25


_NPAD = 10240
_FROWS = _NPAD // _DIM


def _occupancy_body(src_hbm, out_hbm, idx_v, flags_v, sem):
    cid = lax.axis_index("c")
    sid = lax.axis_index("s")
    wid = sid * _NC + cid
    base = wid * _EPW
    cp = pltpu.async_copy(src_hbm.at[pl.ds(base, _EPW)], idx_v, sem)

    zero = jnp.zeros((_LANES,), jnp.float32)

    def zero_body(r, carry):
        for u in range(_DIM // _LANES):
            flags_v[r, pl.ds(u * _LANES, _LANES)] = zero
        return carry

    lax.fori_loop(0, _FROWS, zero_body, 0)
    cp.wait()

    one = jnp.ones((_LANES,), jnp.float32)
    step = _LANES * _UNROLL

    def scatter_body(i, carry):
        for u in range(_UNROLL):
            idx = idx_v[pl.ds(i * step + u * _LANES, _LANES)]
            plsc.store_scatter(flags_v, [idx >> 7, idx & 127], one)
        return carry

    lax.fori_loop(0, _EPW // step, scatter_body, 0)

    pltpu.sync_copy(flags_v, out_hbm.at[pl.ds(wid * _FROWS, _FROWS), :])


@functools.lru_cache(maxsize=None)
def _get_occupancy():
    return pl.kernel(
        _occupancy_body,
        out_type=jax.ShapeDtypeStruct((_NW * _FROWS, _DIM), jnp.float32),
        mesh=plsc.VectorSubcoreMesh(core_axis_name="c", subcore_axis_name="s"),
        scratch_types=[
            pltpu.VMEM((_EPW,), jnp.int32),
            pltpu.VMEM((_FROWS, _DIM), jnp.float32),
            pltpu.SemaphoreType.DMA,
        ],
        compiler_params=pltpu.CompilerParams(
            use_tc_tiling_on_sc=False, needs_layout_passes=False
        ),
    )


_BN = 2000


def _project_body(e_ref, w_ref, b_ref, o_ref):
    rows = e_ref[:]
    nrm = jnp.sqrt(jnp.sum(rows * rows, axis=1, keepdims=True))
    scale = jnp.where(nrm > 1.0, 1.0 / (nrm + 1e-7), 1.0)
    h = rows * scale
    c = lax.dot_general(
        h, w_ref[:], (((1,), (1,)), ((), ())),
        preferred_element_type=jnp.float32,
    ) + b_ref[:]
    o_ref[:] = jnp.maximum(c, 0.0)


_BM = 1024
_BMG = _BM // _DIM


def _mask_body(c_ref, f_ref, o_ref):
    i = pl.program_id(0)
    occ = jnp.zeros((_BMG, _DIM), jnp.float32)
    for w in range(_NW):
        occ = occ + f_ref[pl.ds(w * _FROWS + i * _BMG, _BMG), :]
    occ3 = jnp.transpose(occ.reshape(_BMG, 1, _DIM), (0, 2, 1))
    c3 = c_ref[:].reshape(_BMG, _DIM, _DIM)
    o_ref[:] = jnp.where(occ3 > 0.0, c3, 0.0).reshape(_BM, _DIM)


def kernel(triplets, ent_embed, W_a, b_a, W_a2, b_a2):
    src = triplets[:, 0].astype(jnp.int32)
    flags = _get_occupancy()(src)
    relu_c = pl.pallas_call(
        _project_body,
        grid=(_N // _BN,),
        in_specs=[
            pl.BlockSpec((_BN, _DIM), lambda i: (i, 0)),
            pl.BlockSpec((_DIM, _DIM), lambda i: (0, 0)),
            pl.BlockSpec((1, _DIM), lambda i: (0, 0)),
        ],
        out_specs=pl.BlockSpec((_BN, _DIM), lambda i: (i, 0)),
        out_shape=jax.ShapeDtypeStruct((_N, _DIM), jnp.float32),
    )(ent_embed, W_a, b_a.reshape(1, _DIM))
    out = pl.pallas_call(
        _mask_body,
        grid=(pl.cdiv(_N, _BM),),
        in_specs=[
            pl.BlockSpec((_BM, _DIM), lambda i: (i, 0)),
            pl.BlockSpec((_NW * _FROWS, _DIM), lambda i: (0, 0)),
        ],
        out_specs=pl.BlockSpec((_BM, _DIM), lambda i: (i, 0)),
        out_shape=jax.ShapeDtypeStruct((_N, _DIM), jnp.float32),
    )(relu_c, flags)
    return out


# speedup vs baseline: 1.0460x; 1.0460x over previous
import functools

import jax
import jax.numpy as jnp
from jax import lax
from jax.experimental import pallas as pl
from jax.experimental.pallas import tpu as pltpu
from jax.experimental.pallas import tpu_sc as plsc

_N = 10000
_E = 320000
_DIM = 128
_LANES = 16
_NC = 2
_NS = 16
_NW = _NC * _NS
_EPW = _E // _NW


_UNROLL = 25


_NPAD = 10240
_FROWS = _NPAD // _DIM


def _occupancy_body(src_hbm, out_hbm, idx_v, flags_v, sem):
    cid = lax.axis_index("c")
    sid = lax.axis_index("s")
    wid = sid * _NC + cid
    base = wid * _EPW
    cp = pltpu.async_copy(src_hbm.at[pl.ds(base, _EPW)], idx_v, sem)

    zero = jnp.zeros((_LANES,), jnp.float32)

    def zero_body(i, carry):
        for u in range(32):
            flags_v[pl.ds((i * 32 + u) * _LANES, _LANES)] = zero
        return carry

    lax.fori_loop(0, _NPAD // (_LANES * 32), zero_body, 0)
    cp.wait()

    one = jnp.ones((_LANES,), jnp.float32)
    step = _LANES * _UNROLL

    def scatter_body(i, carry):
        for u in range(_UNROLL):
            idx = idx_v[pl.ds(i * step + u * _LANES, _LANES)]
            plsc.store_scatter(flags_v, [idx], one)
        return carry

    lax.fori_loop(0, _EPW // step, scatter_body, 0)

    pltpu.sync_copy(flags_v, out_hbm.at[wid])


@functools.lru_cache(maxsize=None)
def _get_occupancy():
    return pl.kernel(
        _occupancy_body,
        out_type=jax.ShapeDtypeStruct((_NW, _NPAD), jnp.float32),
        mesh=plsc.VectorSubcoreMesh(core_axis_name="c", subcore_axis_name="s"),
        scratch_types=[
            pltpu.VMEM((_EPW,), jnp.int32),
            pltpu.VMEM((_NPAD,), jnp.float32),
            pltpu.SemaphoreType.DMA,
        ],
        compiler_params=pltpu.CompilerParams(
            use_tc_tiling_on_sc=False, needs_layout_passes=False
        ),
    )


_BN = 2000


def _project_body(e_ref, w_ref, b_ref, o_ref):
    rows = e_ref[:]
    nrm = jnp.sqrt(jnp.sum(rows * rows, axis=1, keepdims=True))
    scale = jnp.where(nrm > 1.0, 1.0 / (nrm + 1e-7), 1.0)
    h = rows * scale
    c = lax.dot_general(
        h, w_ref[:], (((1,), (1,)), ((), ())),
        preferred_element_type=jnp.float32,
    ) + b_ref[:]
    o_ref[:] = jnp.maximum(c, 0.0)


_BM = 2048
_BMG = _BM // _DIM


def _mask_body(c_ref, f_ref, o_ref):
    i = pl.program_id(0)
    occ = jnp.zeros((_BMG, _DIM), jnp.float32)
    for w in range(_NW):
        occ = occ + f_ref[pl.ds(w * _FROWS + i * _BMG, _BMG), :]
    occ3 = jnp.transpose(occ.reshape(_BMG, 1, _DIM), (0, 2, 1))
    c3 = c_ref[:].reshape(_BMG, _DIM, _DIM)
    o_ref[:] = jnp.where(occ3 > 0.0, c3, 0.0).reshape(_BM, _DIM)


def kernel(triplets, ent_embed, W_a, b_a, W_a2, b_a2):
    src = triplets[:, 0].astype(jnp.int32)
    flags = _get_occupancy()(src).reshape(_NW * _FROWS, _DIM)
    relu_c = pl.pallas_call(
        _project_body,
        grid=(_N // _BN,),
        in_specs=[
            pl.BlockSpec((_BN, _DIM), lambda i: (i, 0)),
            pl.BlockSpec((_DIM, _DIM), lambda i: (0, 0)),
            pl.BlockSpec((1, _DIM), lambda i: (0, 0)),
        ],
        out_specs=pl.BlockSpec((_BN, _DIM), lambda i: (i, 0)),
        out_shape=jax.ShapeDtypeStruct((_N, _DIM), jnp.float32),
    )(ent_embed, W_a, b_a.reshape(1, _DIM))
    out = pl.pallas_call(
        _mask_body,
        grid=(pl.cdiv(_N, _BM),),
        in_specs=[
            pl.BlockSpec((_BM, _DIM), lambda i: (i, 0)),
            pl.BlockSpec((_NW * _FROWS, _DIM), lambda i: (0, 0)),
        ],
        out_specs=pl.BlockSpec((_BM, _DIM), lambda i: (i, 0)),
        out_shape=jax.ShapeDtypeStruct((_N, _DIM), jnp.float32),
    )(relu_c, flags)
    return out


# speedup vs baseline: 1.0496x; 1.0035x over previous
import functools

import jax
import jax.numpy as jnp
from jax import lax
from jax.experimental import pallas as pl
from jax.experimental.pallas import tpu as pltpu
from jax.experimental.pallas import tpu_sc as plsc

_N = 10000
_E = 320000
_DIM = 128
_LANES = 16
_NC = 2
_NS = 16
_NW = _NC * _NS
_EPW = _E // _NW


_UNROLL = 25


_NPAD = 10240
_FROWS = _NPAD // _DIM


def _occupancy_body(src_hbm, out_hbm, idx_v, flags_v, sem):
    cid = lax.axis_index("c")
    sid = lax.axis_index("s")
    wid = sid * _NC + cid
    base = wid * _EPW
    cp = pltpu.async_copy(src_hbm.at[pl.ds(base, _EPW)], idx_v, sem)

    zero = jnp.zeros((_LANES,), jnp.float32)

    def zero_body(i, carry):
        for u in range(32):
            flags_v[pl.ds((i * 32 + u) * _LANES, _LANES)] = zero
        return carry

    lax.fori_loop(0, _NPAD // (_LANES * 32), zero_body, 0)
    cp.wait()

    one = jnp.ones((_LANES,), jnp.float32)
    step = _LANES * _UNROLL

    def scatter_body(i, carry):
        for u in range(_UNROLL):
            idx = idx_v[pl.ds(i * step + u * _LANES, _LANES)]
            plsc.store_scatter(flags_v, [idx], one)
        return carry

    lax.fori_loop(0, _EPW // step, scatter_body, 0)

    pltpu.sync_copy(flags_v, out_hbm.at[wid])


@functools.lru_cache(maxsize=None)
def _get_occupancy():
    return pl.kernel(
        _occupancy_body,
        out_type=jax.ShapeDtypeStruct((_NW, _NPAD), jnp.float32),
        mesh=plsc.VectorSubcoreMesh(core_axis_name="c", subcore_axis_name="s"),
        scratch_types=[
            pltpu.VMEM((_EPW,), jnp.int32),
            pltpu.VMEM((_NPAD,), jnp.float32),
            pltpu.SemaphoreType.DMA,
        ],
        compiler_params=pltpu.CompilerParams(
            use_tc_tiling_on_sc=False, needs_layout_passes=False
        ),
    )


_BN = 2000


def _project_body(e_ref, w_ref, b_ref, o_ref):
    rows = e_ref[:]
    nrm = jnp.sqrt(jnp.sum(rows * rows, axis=1, keepdims=True))
    scale = jnp.where(nrm > 1.0, 1.0 / (nrm + 1e-7), 1.0)
    h = rows * scale
    c = lax.dot_general(
        h, w_ref[:], (((1,), (1,)), ((), ())),
        preferred_element_type=jnp.float32,
    ) + b_ref[:]
    o_ref[:] = jnp.maximum(c, 0.0)


_BM = 2048
_BMG = _BM // _DIM


def _mask_body(c_ref, f_ref, o_ref):
    i = pl.program_id(0)
    occ = jnp.zeros((_BMG, _DIM), jnp.float32)
    for w in range(_NW):
        occ = occ + f_ref[pl.ds(w * _FROWS + i * _BMG, _BMG), :]
    occ_t = jnp.transpose(occ)
    for t in range(_BMG):
        rows = pl.ds(t * _DIM, _DIM)
        o_ref[rows, :] = jnp.where(
            occ_t[:, t : t + 1] > 0.0, c_ref[rows, :], 0.0
        )


def kernel(triplets, ent_embed, W_a, b_a, W_a2, b_a2):
    src = triplets[:, 0].astype(jnp.int32)
    flags = _get_occupancy()(src).reshape(_NW * _FROWS, _DIM)
    relu_c = pl.pallas_call(
        _project_body,
        grid=(_N // _BN,),
        in_specs=[
            pl.BlockSpec((_BN, _DIM), lambda i: (i, 0)),
            pl.BlockSpec((_DIM, _DIM), lambda i: (0, 0)),
            pl.BlockSpec((1, _DIM), lambda i: (0, 0)),
        ],
        out_specs=pl.BlockSpec((_BN, _DIM), lambda i: (i, 0)),
        out_shape=jax.ShapeDtypeStruct((_N, _DIM), jnp.float32),
    )(ent_embed, W_a, b_a.reshape(1, _DIM))
    out = pl.pallas_call(
        _mask_body,
        grid=(pl.cdiv(_N, _BM),),
        in_specs=[
            pl.BlockSpec((_BM, _DIM), lambda i: (i, 0)),
            pl.BlockSpec((_NW * _FROWS, _DIM), lambda i: (0, 0)),
        ],
        out_specs=pl.BlockSpec((_BM, _DIM), lambda i: (i, 0)),
        out_shape=jax.ShapeDtypeStruct((_N, _DIM), jnp.float32),
    )(relu_c, flags)
    return out
